# baseline (speedup 1.0000x reference)
import jax
import jax.numpy as jnp
from jax import lax
from jax.experimental import pallas as pl
from jax.experimental.pallas import tpu as pltpu
from jax.experimental.pallas import tpu_sc as plsc

B = 16384
L = 50
D = 32
V = 1000000
NL = 16
NACC = 4

_info = plsc.get_sparse_core_info()
NC = _info.num_cores
NS = _info.num_subcores
NW = NC * NS
B_PER_W = B // NW
CH = 32
N_CHUNKS = B_PER_W // CH

BLK = 128
N_FULL_BLK = V // BLK
TAIL = V - N_FULL_BLK * BLK
BLK_PER_W = (N_FULL_BLK + NW - 1) // NW


def _transpose_body(tableT_hbm, flat_hbm, inb, outb, tin, tout, sem_i, sem_o):
    wid = lax.axis_index("s") * NC + lax.axis_index("c")
    col_iota = lax.iota(jnp.int32, NL) * D

    def do_block(blk, inbuf, outbuf, n_rows):
        pltpu.sync_copy(tableT_hbm.at[:, pl.ds(blk * BLK, n_rows)], inbuf)

        def d_body(d, carry):
            for k in range(n_rows // NL):
                x = inbuf[d, pl.ds(k * NL, NL)]
                idx = col_iota + (k * NL * D + d)
                plsc.store_scatter(outbuf, [idx], x)
            return carry

        lax.fori_loop(0, D, d_body, 0)
        pltpu.sync_copy(outbuf, flat_hbm.at[pl.ds(blk * BLK * D, n_rows * D)])

    def loop_body(k, carry):
        blk = wid + k * NW

        @pl.when(blk < N_FULL_BLK)
        def _():
            do_block(blk, inb, outb, BLK)

        return carry

    lax.fori_loop(0, BLK_PER_W, loop_body, 0)

    @pl.when(wid == NW - 1)
    def _():
        do_block(N_FULL_BLK, tin, tout, TAIL)


def _gather_body(labels_hbm, table_hbm, out_hbm,
                 idx0, idx1, rows0, rows1, outv0, outv1,
                 gsem0, gsem1, osem0, osem1):
    wid = lax.axis_index("s") * NC + lax.axis_index("c")
    base_row = wid * B_PER_W
    scale = jnp.float32(1.0 / L)
    idx_v = (idx0, idx1)
    rows_v = (rows0, rows1)
    out_v = (outv0, outv1)
    gsem = (gsem0, gsem1)
    osem = (osem0, osem1)

    def stage_and_fire(c, b):
        row0 = base_row + c * CH
        pltpu.sync_copy(labels_hbm.at[pl.ds(row0 * L, CH * L)], idx_v[b])
        pltpu.async_copy(table_hbm.at[idx_v[b]], rows_v[b], gsem[b])

    def compute_chunk(c, b):
        row0 = base_row + c * CH
        pltpu.make_async_copy(table_hbm.at[idx_v[b]], rows_v[b], gsem[b]).wait()

        def row_body(r, carry):
            g0 = r * L
            acc = [jnp.zeros((NL,), jnp.float32) for _ in range(2 * NACC)]
            for j in range(L):
                k = j % NACC
                acc[k] = acc[k] + rows_v[b][g0 + j, pl.ds(0, NL)]
                acc[NACC + k] = acc[NACC + k] + rows_v[b][g0 + j, pl.ds(NL, NL)]
            lo = (acc[0] + acc[1]) + (acc[2] + acc[3])
            hi = (acc[4] + acc[5]) + (acc[6] + acc[7])
            out_v[b][r, pl.ds(0, NL)] = lo * scale
            out_v[b][r, pl.ds(NL, NL)] = hi * scale
            return carry

        lax.fori_loop(0, CH, row_body, 0)
        pltpu.async_copy(out_v[b], out_hbm.at[pl.ds(row0, CH)], osem[b])

    stage_and_fire(0, 0)

    def loop_body(c2, carry):
        for bb in range(2):
            c = c2 * 2 + bb

            @pl.when(c + 1 < N_CHUNKS)
            def _():
                stage_and_fire(c + 1, 1 - bb)

            @pl.when(c >= 2)
            def _():
                pltpu.make_async_copy(
                    out_v[bb], out_hbm.at[pl.ds(base_row + c * CH, CH)],
                    osem[bb]).wait()

            compute_chunk(c, bb)
        return carry

    lax.fori_loop(0, N_CHUNKS // 2, loop_body, 0)

    for bb in range(2):
        pltpu.make_async_copy(
            out_v[bb], out_hbm.at[pl.ds(base_row, CH)], osem[bb]).wait()


def kernel(label_lists, table):
    mesh = plsc.VectorSubcoreMesh(core_axis_name="c", subcore_axis_name="s")

    transpose_k = pl.kernel(
        _transpose_body,
        mesh=mesh,
        out_type=jax.ShapeDtypeStruct((V * D,), jnp.float32),
        scratch_types=[
            pltpu.VMEM((D, BLK), jnp.float32),
            pltpu.VMEM((BLK * D,), jnp.float32),
            pltpu.VMEM((D, TAIL), jnp.float32),
            pltpu.VMEM((TAIL * D,), jnp.float32),
            pltpu.SemaphoreType.DMA,
            pltpu.SemaphoreType.DMA,
        ],
        compiler_params=pltpu.CompilerParams(
            use_tc_tiling_on_sc=True, needs_layout_passes=False),
    )
    flat_table = transpose_k(table.T)
    table_rm = flat_table.reshape(V, D)

    labels_flat = label_lists.reshape(-1).astype(jnp.int32)
    gather_k = pl.kernel(
        _gather_body,
        mesh=mesh,
        out_type=jax.ShapeDtypeStruct((B, D), jnp.float32),
        scratch_types=[
            pltpu.VMEM((CH * L,), jnp.int32),
            pltpu.VMEM((CH * L,), jnp.int32),
            pltpu.VMEM((CH * L, D), jnp.float32),
            pltpu.VMEM((CH * L, D), jnp.float32),
            pltpu.VMEM((CH, D), jnp.float32),
            pltpu.VMEM((CH, D), jnp.float32),
            pltpu.SemaphoreType.DMA,
            pltpu.SemaphoreType.DMA,
            pltpu.SemaphoreType.DMA,
            pltpu.SemaphoreType.DMA,
        ],
        compiler_params=pltpu.CompilerParams(use_tc_tiling_on_sc=False),
    )
    return gather_k(labels_flat, table_rm)


# speedup vs baseline: 1.3611x; 1.3611x over previous
import jax
import jax.numpy as jnp
from jax import lax
from jax.experimental import pallas as pl
from jax.experimental.pallas import tpu as pltpu
from jax.experimental.pallas import tpu_sc as plsc

B = 16384
L = 50
D = 32
V = 1000000
NL = 16
NACC = 4

_info = plsc.get_sparse_core_info()
NC = _info.num_cores
NS = _info.num_subcores
NW = NC * NS
B_PER_W = B // NW
CH = 32
N_CHUNKS = B_PER_W // CH

SB = 512
N_FULL_SB = V // SB
TAIL = V - N_FULL_SB * SB
SB_PER_W = (N_FULL_SB + NW - 1) // NW


def _transpose_body(tableT_hbm, flat_hbm,
                    in0, in1, out0, out1, tin, tout,
                    isem0, isem1, osem0, osem1):
    wid = lax.axis_index("s") * NC + lax.axis_index("c")
    col_iota = lax.iota(jnp.int32, NL) * D
    inb = (in0, in1)
    outb = (out0, out1)
    isem = (isem0, isem1)
    osem = (osem0, osem1)

    def sb_of(k):
        return wid + k * NW

    def fire_in(k, b):
        @pl.when(sb_of(k) < N_FULL_SB)
        def _():
            pltpu.async_copy(
                tableT_hbm.at[:, pl.ds(sb_of(k) * SB, SB)], inb[b], isem[b])

    def transpose_into(inbuf, outbuf, n_rows):
        def d_body(d, carry):
            for k in range(n_rows // NL):
                x = inbuf[d, pl.ds(k * NL, NL)]
                idx = col_iota + (k * NL * D + d)
                plsc.store_scatter(outbuf, [idx], x)
            return carry

        lax.fori_loop(0, D, d_body, 0)

    fire_in(0, 0)

    def loop_body(k2, carry):
        for bb in range(2):
            k = k2 * 2 + bb
            fire_in(k + 1, 1 - bb)

            @pl.when(sb_of(k) < N_FULL_SB)
            def _():
                pltpu.make_async_copy(
                    tableT_hbm.at[:, pl.ds(sb_of(k) * SB, SB)],
                    inb[bb], isem[bb]).wait()

                @pl.when(k >= 2)
                def _():
                    pltpu.make_async_copy(
                        outb[bb],
                        flat_hbm.at[pl.ds(sb_of(k) * SB * D, SB * D)],
                        osem[bb]).wait()

                transpose_into(inb[bb], outb[bb], SB)
                pltpu.async_copy(
                    outb[bb],
                    flat_hbm.at[pl.ds(sb_of(k) * SB * D, SB * D)], osem[bb])
        return carry

    lax.fori_loop(0, SB_PER_W // 2, loop_body, 0)

    for bb in range(2):
        pltpu.make_async_copy(
            outb[bb], flat_hbm.at[pl.ds(0, SB * D)], osem[bb]).wait()

    @pl.when(wid == NW - 1)
    def _():
        pltpu.sync_copy(tableT_hbm.at[:, pl.ds(N_FULL_SB * SB, TAIL)], tin)
        transpose_into(tin, tout, TAIL)
        pltpu.sync_copy(tout, flat_hbm.at[pl.ds(N_FULL_SB * SB * D, TAIL * D)])


def _gather_body(labels_hbm, table_hbm, out_hbm,
                 idx0, idx1, rows0, rows1, outv0, outv1,
                 gsem0, gsem1, osem0, osem1):
    wid = lax.axis_index("s") * NC + lax.axis_index("c")
    base_row = wid * B_PER_W
    scale = jnp.float32(1.0 / L)
    idx_v = (idx0, idx1)
    rows_v = (rows0, rows1)
    out_v = (outv0, outv1)
    gsem = (gsem0, gsem1)
    osem = (osem0, osem1)

    def stage_and_fire(c, b):
        row0 = base_row + c * CH
        pltpu.sync_copy(labels_hbm.at[pl.ds(row0 * L, CH * L)], idx_v[b])
        pltpu.async_copy(table_hbm.at[idx_v[b]], rows_v[b], gsem[b])

    def compute_chunk(c, b):
        row0 = base_row + c * CH
        pltpu.make_async_copy(table_hbm.at[idx_v[b]], rows_v[b], gsem[b]).wait()

        def row_body(r, carry):
            g0 = r * L
            acc = [jnp.zeros((NL,), jnp.float32) for _ in range(2 * NACC)]
            for j in range(L):
                k = j % NACC
                acc[k] = acc[k] + rows_v[b][g0 + j, pl.ds(0, NL)]
                acc[NACC + k] = acc[NACC + k] + rows_v[b][g0 + j, pl.ds(NL, NL)]
            lo = (acc[0] + acc[1]) + (acc[2] + acc[3])
            hi = (acc[4] + acc[5]) + (acc[6] + acc[7])
            out_v[b][r, pl.ds(0, NL)] = lo * scale
            out_v[b][r, pl.ds(NL, NL)] = hi * scale
            return carry

        lax.fori_loop(0, CH, row_body, 0)
        pltpu.async_copy(out_v[b], out_hbm.at[pl.ds(row0, CH)], osem[b])

    stage_and_fire(0, 0)

    def loop_body(c2, carry):
        for bb in range(2):
            c = c2 * 2 + bb

            @pl.when(c + 1 < N_CHUNKS)
            def _():
                stage_and_fire(c + 1, 1 - bb)

            @pl.when(c >= 2)
            def _():
                pltpu.make_async_copy(
                    out_v[bb], out_hbm.at[pl.ds(base_row + c * CH, CH)],
                    osem[bb]).wait()

            compute_chunk(c, bb)
        return carry

    lax.fori_loop(0, N_CHUNKS // 2, loop_body, 0)

    for bb in range(2):
        pltpu.make_async_copy(
            out_v[bb], out_hbm.at[pl.ds(base_row, CH)], osem[bb]).wait()


def kernel(label_lists, table):
    mesh = plsc.VectorSubcoreMesh(core_axis_name="c", subcore_axis_name="s")

    transpose_k = pl.kernel(
        _transpose_body,
        mesh=mesh,
        out_type=jax.ShapeDtypeStruct((V * D,), jnp.float32),
        scratch_types=[
            pltpu.VMEM((D, SB), jnp.float32),
            pltpu.VMEM((D, SB), jnp.float32),
            pltpu.VMEM((SB * D,), jnp.float32),
            pltpu.VMEM((SB * D,), jnp.float32),
            pltpu.VMEM((D, TAIL), jnp.float32),
            pltpu.VMEM((TAIL * D,), jnp.float32),
            pltpu.SemaphoreType.DMA,
            pltpu.SemaphoreType.DMA,
            pltpu.SemaphoreType.DMA,
            pltpu.SemaphoreType.DMA,
        ],
        compiler_params=pltpu.CompilerParams(
            use_tc_tiling_on_sc=True, needs_layout_passes=False),
    )
    flat_table = transpose_k(table.T)
    table_rm = flat_table.reshape(V, D)

    labels_flat = label_lists.reshape(-1).astype(jnp.int32)
    gather_k = pl.kernel(
        _gather_body,
        mesh=mesh,
        out_type=jax.ShapeDtypeStruct((B, D), jnp.float32),
        scratch_types=[
            pltpu.VMEM((CH * L,), jnp.int32),
            pltpu.VMEM((CH * L,), jnp.int32),
            pltpu.VMEM((CH * L, D), jnp.float32),
            pltpu.VMEM((CH * L, D), jnp.float32),
            pltpu.VMEM((CH, D), jnp.float32),
            pltpu.VMEM((CH, D), jnp.float32),
            pltpu.SemaphoreType.DMA,
            pltpu.SemaphoreType.DMA,
            pltpu.SemaphoreType.DMA,
            pltpu.SemaphoreType.DMA,
        ],
        compiler_params=pltpu.CompilerParams(use_tc_tiling_on_sc=False),
    )
    return gather_k(labels_flat, table_rm)


# speedup vs baseline: 1.8195x; 1.3367x over previous
import jax
import jax.numpy as jnp
from jax import lax
from jax.experimental import pallas as pl
from jax.experimental.pallas import tpu as pltpu
from jax.experimental.pallas import tpu_sc as plsc

B = 16384
L = 50
D = 32
V = 1000000
NL = 16
NACC = 4

_info = plsc.get_sparse_core_info()
NC = _info.num_cores
NS = _info.num_subcores
NW = NC * NS
B_PER_W = B // NW
CH = 32
N_CHUNKS = B_PER_W // CH

SB = 512
N_FULL_SB = V // SB
TAIL = V - N_FULL_SB * SB
SB_PER_W = (N_FULL_SB + NW - 1) // NW


def _transpose_body(tableT_hbm, flat_hbm,
                    in0, in1, out0, out1, tin, tout, mini0, mini1,
                    isem0, isem1, osem0, osem1):
    wid = lax.axis_index("s") * NC + lax.axis_index("c")
    iota17 = lax.iota(jnp.int32, NL) * (NL + 1)
    inb = (in0, in1)
    outb = (out0, out1)
    mini = (mini0, mini1)
    isem = (isem0, isem1)
    osem = (osem0, osem1)

    def sb_of(k):
        return wid + k * NW

    def fire_in(k, b):
        @pl.when(sb_of(k) < N_FULL_SB)
        def _():
            pltpu.async_copy(
                tableT_hbm.at[:, pl.ds(sb_of(k) * SB, SB)], inb[b], isem[b])

    def transpose_into(inbuf, outbuf, n_rows):
        def c_body(cc, carry):
            c0 = cc * NL
            for dr in range(2):
                for dd in range(NL):
                    x = inbuf[dr * NL + dd, pl.ds(c0, NL)]
                    plsc.store_scatter(mini[dr], [iota17 + dd], x)
            for c in range(NL):
                base = (c0 + c) * D
                outbuf[pl.ds(base, NL)] = mini[0][pl.ds(c * (NL + 1), NL)]
                outbuf[pl.ds(base + NL, NL)] = mini[1][pl.ds(c * (NL + 1), NL)]
            return carry

        lax.fori_loop(0, n_rows // NL, c_body, 0)

    fire_in(0, 0)

    def loop_body(k2, carry):
        for bb in range(2):
            k = k2 * 2 + bb
            fire_in(k + 1, 1 - bb)

            @pl.when(sb_of(k) < N_FULL_SB)
            def _():
                pltpu.make_async_copy(
                    tableT_hbm.at[:, pl.ds(sb_of(k) * SB, SB)],
                    inb[bb], isem[bb]).wait()

                @pl.when(k >= 2)
                def _():
                    pltpu.make_async_copy(
                        outb[bb],
                        flat_hbm.at[pl.ds(sb_of(k) * SB * D, SB * D)],
                        osem[bb]).wait()

                transpose_into(inb[bb], outb[bb], SB)
                pltpu.async_copy(
                    outb[bb],
                    flat_hbm.at[pl.ds(sb_of(k) * SB * D, SB * D)], osem[bb])
        return carry

    lax.fori_loop(0, SB_PER_W // 2, loop_body, 0)

    for bb in range(2):
        pltpu.make_async_copy(
            outb[bb], flat_hbm.at[pl.ds(0, SB * D)], osem[bb]).wait()

    @pl.when(wid == NW - 1)
    def _():
        pltpu.sync_copy(tableT_hbm.at[:, pl.ds(N_FULL_SB * SB, TAIL)], tin)
        transpose_into(tin, tout, TAIL)
        pltpu.sync_copy(tout, flat_hbm.at[pl.ds(N_FULL_SB * SB * D, TAIL * D)])


def _gather_body(labels_hbm, table_hbm, out_hbm,
                 idx0, idx1, rows0, rows1, outv0, outv1,
                 gsem0, gsem1, osem0, osem1):
    wid = lax.axis_index("s") * NC + lax.axis_index("c")
    base_row = wid * B_PER_W
    scale = jnp.float32(1.0 / L)
    idx_v = (idx0, idx1)
    rows_v = (rows0, rows1)
    out_v = (outv0, outv1)
    gsem = (gsem0, gsem1)
    osem = (osem0, osem1)

    def stage_and_fire(c, b):
        row0 = base_row + c * CH
        pltpu.sync_copy(labels_hbm.at[pl.ds(row0 * L, CH * L)], idx_v[b])
        pltpu.async_copy(table_hbm.at[idx_v[b]], rows_v[b], gsem[b])

    def compute_chunk(c, b):
        row0 = base_row + c * CH
        pltpu.make_async_copy(table_hbm.at[idx_v[b]], rows_v[b], gsem[b]).wait()

        def row_body(r, carry):
            g0 = r * L
            acc = [jnp.zeros((NL,), jnp.float32) for _ in range(2 * NACC)]
            for j in range(L):
                k = j % NACC
                acc[k] = acc[k] + rows_v[b][g0 + j, pl.ds(0, NL)]
                acc[NACC + k] = acc[NACC + k] + rows_v[b][g0 + j, pl.ds(NL, NL)]
            lo = (acc[0] + acc[1]) + (acc[2] + acc[3])
            hi = (acc[4] + acc[5]) + (acc[6] + acc[7])
            out_v[b][r, pl.ds(0, NL)] = lo * scale
            out_v[b][r, pl.ds(NL, NL)] = hi * scale
            return carry

        lax.fori_loop(0, CH, row_body, 0)
        pltpu.async_copy(out_v[b], out_hbm.at[pl.ds(row0, CH)], osem[b])

    stage_and_fire(0, 0)

    def loop_body(c2, carry):
        for bb in range(2):
            c = c2 * 2 + bb

            @pl.when(c + 1 < N_CHUNKS)
            def _():
                stage_and_fire(c + 1, 1 - bb)

            @pl.when(c >= 2)
            def _():
                pltpu.make_async_copy(
                    out_v[bb], out_hbm.at[pl.ds(base_row + c * CH, CH)],
                    osem[bb]).wait()

            compute_chunk(c, bb)
        return carry

    lax.fori_loop(0, N_CHUNKS // 2, loop_body, 0)

    for bb in range(2):
        pltpu.make_async_copy(
            out_v[bb], out_hbm.at[pl.ds(base_row, CH)], osem[bb]).wait()


def kernel(label_lists, table):
    mesh = plsc.VectorSubcoreMesh(core_axis_name="c", subcore_axis_name="s")

    transpose_k = pl.kernel(
        _transpose_body,
        mesh=mesh,
        out_type=jax.ShapeDtypeStruct((V * D,), jnp.float32),
        scratch_types=[
            pltpu.VMEM((D, SB), jnp.float32),
            pltpu.VMEM((D, SB), jnp.float32),
            pltpu.VMEM((SB * D,), jnp.float32),
            pltpu.VMEM((SB * D,), jnp.float32),
            pltpu.VMEM((D, TAIL), jnp.float32),
            pltpu.VMEM((TAIL * D,), jnp.float32),
            pltpu.VMEM((NL * (NL + 1),), jnp.float32),
            pltpu.VMEM((NL * (NL + 1),), jnp.float32),
            pltpu.SemaphoreType.DMA,
            pltpu.SemaphoreType.DMA,
            pltpu.SemaphoreType.DMA,
            pltpu.SemaphoreType.DMA,
        ],
        compiler_params=pltpu.CompilerParams(
            use_tc_tiling_on_sc=True, needs_layout_passes=False),
    )
    flat_table = transpose_k(table.T)
    table_rm = flat_table.reshape(V, D)

    labels_flat = label_lists.reshape(-1).astype(jnp.int32)
    gather_k = pl.kernel(
        _gather_body,
        mesh=mesh,
        out_type=jax.ShapeDtypeStruct((B, D), jnp.float32),
        scratch_types=[
            pltpu.VMEM((CH * L,), jnp.int32),
            pltpu.VMEM((CH * L,), jnp.int32),
            pltpu.VMEM((CH * L, D), jnp.float32),
            pltpu.VMEM((CH * L, D), jnp.float32),
            pltpu.VMEM((CH, D), jnp.float32),
            pltpu.VMEM((CH, D), jnp.float32),
            pltpu.SemaphoreType.DMA,
            pltpu.SemaphoreType.DMA,
            pltpu.SemaphoreType.DMA,
            pltpu.SemaphoreType.DMA,
        ],
        compiler_params=pltpu.CompilerParams(use_tc_tiling_on_sc=False),
    )
    return gather_k(labels_flat, table_rm)


# speedup vs baseline: 1.8207x; 1.0006x over previous
import jax
import jax.numpy as jnp
from jax import lax
from jax.experimental import pallas as pl
from jax.experimental.pallas import tpu as pltpu
from jax.experimental.pallas import tpu_sc as plsc

B = 16384
L = 50
D = 32
V = 1000000
NL = 16
NACC = 4

_info = plsc.get_sparse_core_info()
NC = _info.num_cores
NS = _info.num_subcores
NW = NC * NS
B_PER_W = B // NW
CH = 32
N_CHUNKS = B_PER_W // CH

SB = 512
N_FULL_SB = V // SB
TAIL = V - N_FULL_SB * SB
SB_PER_W = (N_FULL_SB + NW - 1) // NW


def _transpose_body(tableT_hbm, flat_hbm,
                    in0, in1, out0, out1, tin, tout, mini0, mini1, mini2, mini3,
                    isem0, isem1, osem0, osem1):
    wid = lax.axis_index("s") * NC + lax.axis_index("c")
    iota17 = lax.iota(jnp.int32, NL) * (NL + 1)
    inb = (in0, in1)
    outb = (out0, out1)
    mini = (mini0, mini1, mini2, mini3)
    isem = (isem0, isem1)
    osem = (osem0, osem1)

    def sb_of(k):
        return wid + k * NW

    def fire_in(k, b):
        @pl.when(sb_of(k) < N_FULL_SB)
        def _():
            pltpu.async_copy(
                tableT_hbm.at[:, pl.ds(sb_of(k) * SB, SB)], inb[b], isem[b])

    def transpose_into(inbuf, outbuf, n_rows):
        def phase1(c0, mpair):
            for dr in range(2):
                for dd in range(NL):
                    x = inbuf[dr * NL + dd, pl.ds(c0, NL)]
                    plsc.store_scatter(mpair[dr], [iota17 + dd], x)

        def phase2(c0, mpair):
            for c in range(NL):
                base = (c0 + c) * D
                outbuf[pl.ds(base, NL)] = mpair[0][pl.ds(c * (NL + 1), NL)]
                outbuf[pl.ds(base + NL, NL)] = mpair[1][pl.ds(c * (NL + 1), NL)]

        def c_body(cc2, carry):
            cA = cc2 * 2 * NL
            cB = (cc2 * 2 + 1) * NL
            phase1(cA, (mini[0], mini[1]))
            phase1(cB, (mini[2], mini[3]))
            phase2(cA, (mini[0], mini[1]))
            phase2(cB, (mini[2], mini[3]))
            return carry

        lax.fori_loop(0, n_rows // (2 * NL), c_body, 0)

    fire_in(0, 0)

    def loop_body(k2, carry):
        for bb in range(2):
            k = k2 * 2 + bb
            fire_in(k + 1, 1 - bb)

            @pl.when(sb_of(k) < N_FULL_SB)
            def _():
                pltpu.make_async_copy(
                    tableT_hbm.at[:, pl.ds(sb_of(k) * SB, SB)],
                    inb[bb], isem[bb]).wait()

                @pl.when(k >= 2)
                def _():
                    pltpu.make_async_copy(
                        outb[bb],
                        flat_hbm.at[pl.ds(sb_of(k) * SB * D, SB * D)],
                        osem[bb]).wait()

                transpose_into(inb[bb], outb[bb], SB)
                pltpu.async_copy(
                    outb[bb],
                    flat_hbm.at[pl.ds(sb_of(k) * SB * D, SB * D)], osem[bb])
        return carry

    lax.fori_loop(0, SB_PER_W // 2, loop_body, 0)

    for bb in range(2):
        pltpu.make_async_copy(
            outb[bb], flat_hbm.at[pl.ds(0, SB * D)], osem[bb]).wait()

    @pl.when(wid == NW - 1)
    def _():
        pltpu.sync_copy(tableT_hbm.at[:, pl.ds(N_FULL_SB * SB, TAIL)], tin)
        transpose_into(tin, tout, TAIL)
        pltpu.sync_copy(tout, flat_hbm.at[pl.ds(N_FULL_SB * SB * D, TAIL * D)])


def _gather_body(labels_hbm, table_hbm, out_hbm,
                 idx0, idx1, rows0, rows1, outv0, outv1,
                 gsem0, gsem1, osem0, osem1):
    wid = lax.axis_index("s") * NC + lax.axis_index("c")
    base_row = wid * B_PER_W
    scale = jnp.float32(1.0 / L)
    idx_v = (idx0, idx1)
    rows_v = (rows0, rows1)
    out_v = (outv0, outv1)
    gsem = (gsem0, gsem1)
    osem = (osem0, osem1)

    def stage_and_fire(c, b):
        row0 = base_row + c * CH
        pltpu.sync_copy(labels_hbm.at[pl.ds(row0 * L, CH * L)], idx_v[b])
        pltpu.async_copy(table_hbm.at[idx_v[b]], rows_v[b], gsem[b])

    def compute_chunk(c, b):
        row0 = base_row + c * CH
        pltpu.make_async_copy(table_hbm.at[idx_v[b]], rows_v[b], gsem[b]).wait()

        def row_body(r, carry):
            g0 = r * L
            acc = [jnp.zeros((NL,), jnp.float32) for _ in range(2 * NACC)]
            for j in range(L):
                k = j % NACC
                acc[k] = acc[k] + rows_v[b][g0 + j, pl.ds(0, NL)]
                acc[NACC + k] = acc[NACC + k] + rows_v[b][g0 + j, pl.ds(NL, NL)]
            lo = (acc[0] + acc[1]) + (acc[2] + acc[3])
            hi = (acc[4] + acc[5]) + (acc[6] + acc[7])
            out_v[b][r, pl.ds(0, NL)] = lo * scale
            out_v[b][r, pl.ds(NL, NL)] = hi * scale
            return carry

        lax.fori_loop(0, CH, row_body, 0)
        pltpu.async_copy(out_v[b], out_hbm.at[pl.ds(row0, CH)], osem[b])

    stage_and_fire(0, 0)

    def loop_body(c2, carry):
        for bb in range(2):
            c = c2 * 2 + bb

            @pl.when(c + 1 < N_CHUNKS)
            def _():
                stage_and_fire(c + 1, 1 - bb)

            @pl.when(c >= 2)
            def _():
                pltpu.make_async_copy(
                    out_v[bb], out_hbm.at[pl.ds(base_row + c * CH, CH)],
                    osem[bb]).wait()

            compute_chunk(c, bb)
        return carry

    lax.fori_loop(0, N_CHUNKS // 2, loop_body, 0)

    for bb in range(2):
        pltpu.make_async_copy(
            out_v[bb], out_hbm.at[pl.ds(base_row, CH)], osem[bb]).wait()


def kernel(label_lists, table):
    mesh = plsc.VectorSubcoreMesh(core_axis_name="c", subcore_axis_name="s")

    transpose_k = pl.kernel(
        _transpose_body,
        mesh=mesh,
        out_type=jax.ShapeDtypeStruct((V * D,), jnp.float32),
        scratch_types=[
            pltpu.VMEM((D, SB), jnp.float32),
            pltpu.VMEM((D, SB), jnp.float32),
            pltpu.VMEM((SB * D,), jnp.float32),
            pltpu.VMEM((SB * D,), jnp.float32),
            pltpu.VMEM((D, TAIL), jnp.float32),
            pltpu.VMEM((TAIL * D,), jnp.float32),
            pltpu.VMEM((NL * (NL + 1),), jnp.float32),
            pltpu.VMEM((NL * (NL + 1),), jnp.float32),
            pltpu.VMEM((NL * (NL + 1),), jnp.float32),
            pltpu.VMEM((NL * (NL + 1),), jnp.float32),
            pltpu.SemaphoreType.DMA,
            pltpu.SemaphoreType.DMA,
            pltpu.SemaphoreType.DMA,
            pltpu.SemaphoreType.DMA,
        ],
        compiler_params=pltpu.CompilerParams(
            use_tc_tiling_on_sc=True, needs_layout_passes=False),
    )
    flat_table = transpose_k(table.T)
    table_rm = flat_table.reshape(V, D)

    labels_flat = label_lists.reshape(-1).astype(jnp.int32)
    gather_k = pl.kernel(
        _gather_body,
        mesh=mesh,
        out_type=jax.ShapeDtypeStruct((B, D), jnp.float32),
        scratch_types=[
            pltpu.VMEM((CH * L,), jnp.int32),
            pltpu.VMEM((CH * L,), jnp.int32),
            pltpu.VMEM((CH * L, D), jnp.float32),
            pltpu.VMEM((CH * L, D), jnp.float32),
            pltpu.VMEM((CH, D), jnp.float32),
            pltpu.VMEM((CH, D), jnp.float32),
            pltpu.SemaphoreType.DMA,
            pltpu.SemaphoreType.DMA,
            pltpu.SemaphoreType.DMA,
            pltpu.SemaphoreType.DMA,
        ],
        compiler_params=pltpu.CompilerParams(use_tc_tiling_on_sc=False),
    )
    return gather_k(labels_flat, table_rm)


# speedup vs baseline: 2.7744x; 1.5239x over previous
import jax
import jax.numpy as jnp
from jax import lax
from jax.experimental import pallas as pl
from jax.experimental.pallas import tpu as pltpu
from jax.experimental.pallas import tpu_sc as plsc

B = 16384
L = 50
D = 32
V = 1000000
NL = 16
NACC = 4

_info = plsc.get_sparse_core_info()
NC = _info.num_cores
NS = _info.num_subcores
NW = NC * NS
B_PER_W = B // NW
CH = 32
N_CHUNKS = B_PER_W // CH

SB = 512
N_FULL_SB = V // SB
TAIL = V - N_FULL_SB * SB
SB_PER_W = (N_FULL_SB + NW - 1) // NW


def _transpose_body(tableT_hbm, flat_hbm,
                    in0, in1, out0, out1, tin, tout, mini0, mini1, mini2, mini3,
                    isem0, isem1, osem0, osem1):
    wid = lax.axis_index("s") * NC + lax.axis_index("c")
    iota17 = lax.iota(jnp.int32, NL) * (NL + 1)
    inb = (in0, in1)
    outb = (out0, out1)
    mini = (mini0, mini1, mini2, mini3)
    isem = (isem0, isem1)
    osem = (osem0, osem1)

    def sb_of(k):
        return wid + k * NW

    def fire_in(k, b):
        @pl.when(sb_of(k) < N_FULL_SB)
        def _():
            pltpu.async_copy(
                tableT_hbm.at[:, pl.ds(sb_of(k) * SB, SB)], inb[b], isem[b])

    def transpose_into(inbuf, outbuf, n_rows):
        iota = lax.iota(jnp.int32, NL)
        diags = [jnp.bitwise_and(iota + s, NL - 1) for s in range(NL)]

        def c_body(cc, carry):
            c0 = cc * NL
            for dr in range(2):
                row_vec = iota + dr * NL
                for s in range(NL):
                    col = diags[s] + c0
                    x = plsc.load_gather(inbuf, [row_vec, col])
                    oidx = col * D + row_vec
                    plsc.store_scatter(outbuf, [oidx], x)
            return carry

        lax.fori_loop(0, n_rows // NL, c_body, 0)

    fire_in(0, 0)

    def loop_body(k2, carry):
        for bb in range(2):
            k = k2 * 2 + bb
            fire_in(k + 1, 1 - bb)

            @pl.when(sb_of(k) < N_FULL_SB)
            def _():
                pltpu.make_async_copy(
                    tableT_hbm.at[:, pl.ds(sb_of(k) * SB, SB)],
                    inb[bb], isem[bb]).wait()

                @pl.when(k >= 2)
                def _():
                    pltpu.make_async_copy(
                        outb[bb],
                        flat_hbm.at[pl.ds(sb_of(k) * SB * D, SB * D)],
                        osem[bb]).wait()

                transpose_into(inb[bb], outb[bb], SB)
                pltpu.async_copy(
                    outb[bb],
                    flat_hbm.at[pl.ds(sb_of(k) * SB * D, SB * D)], osem[bb])
        return carry

    lax.fori_loop(0, SB_PER_W // 2, loop_body, 0)

    for bb in range(2):
        pltpu.make_async_copy(
            outb[bb], flat_hbm.at[pl.ds(0, SB * D)], osem[bb]).wait()

    @pl.when(wid == NW - 1)
    def _():
        pltpu.sync_copy(tableT_hbm.at[:, pl.ds(N_FULL_SB * SB, TAIL)], tin)
        transpose_into(tin, tout, TAIL)
        pltpu.sync_copy(tout, flat_hbm.at[pl.ds(N_FULL_SB * SB * D, TAIL * D)])


def _gather_body(labels_hbm, table_hbm, out_hbm,
                 idx0, idx1, rows0, rows1, outv0, outv1,
                 gsem0, gsem1, osem0, osem1):
    wid = lax.axis_index("s") * NC + lax.axis_index("c")
    base_row = wid * B_PER_W
    scale = jnp.float32(1.0 / L)
    idx_v = (idx0, idx1)
    rows_v = (rows0, rows1)
    out_v = (outv0, outv1)
    gsem = (gsem0, gsem1)
    osem = (osem0, osem1)

    def stage_and_fire(c, b):
        row0 = base_row + c * CH
        pltpu.sync_copy(labels_hbm.at[pl.ds(row0 * L, CH * L)], idx_v[b])
        pltpu.async_copy(table_hbm.at[idx_v[b]], rows_v[b], gsem[b])

    def compute_chunk(c, b):
        row0 = base_row + c * CH
        pltpu.make_async_copy(table_hbm.at[idx_v[b]], rows_v[b], gsem[b]).wait()

        def row_body(r, carry):
            g0 = r * L
            acc = [jnp.zeros((NL,), jnp.float32) for _ in range(2 * NACC)]
            for j in range(L):
                k = j % NACC
                acc[k] = acc[k] + rows_v[b][g0 + j, pl.ds(0, NL)]
                acc[NACC + k] = acc[NACC + k] + rows_v[b][g0 + j, pl.ds(NL, NL)]
            lo = (acc[0] + acc[1]) + (acc[2] + acc[3])
            hi = (acc[4] + acc[5]) + (acc[6] + acc[7])
            out_v[b][r, pl.ds(0, NL)] = lo * scale
            out_v[b][r, pl.ds(NL, NL)] = hi * scale
            return carry

        lax.fori_loop(0, CH, row_body, 0)
        pltpu.async_copy(out_v[b], out_hbm.at[pl.ds(row0, CH)], osem[b])

    stage_and_fire(0, 0)

    def loop_body(c2, carry):
        for bb in range(2):
            c = c2 * 2 + bb

            @pl.when(c + 1 < N_CHUNKS)
            def _():
                stage_and_fire(c + 1, 1 - bb)

            @pl.when(c >= 2)
            def _():
                pltpu.make_async_copy(
                    out_v[bb], out_hbm.at[pl.ds(base_row + c * CH, CH)],
                    osem[bb]).wait()

            compute_chunk(c, bb)
        return carry

    lax.fori_loop(0, N_CHUNKS // 2, loop_body, 0)

    for bb in range(2):
        pltpu.make_async_copy(
            out_v[bb], out_hbm.at[pl.ds(base_row, CH)], osem[bb]).wait()


def kernel(label_lists, table):
    mesh = plsc.VectorSubcoreMesh(core_axis_name="c", subcore_axis_name="s")

    transpose_k = pl.kernel(
        _transpose_body,
        mesh=mesh,
        out_type=jax.ShapeDtypeStruct((V * D,), jnp.float32),
        scratch_types=[
            pltpu.VMEM((D, SB), jnp.float32),
            pltpu.VMEM((D, SB), jnp.float32),
            pltpu.VMEM((SB * D,), jnp.float32),
            pltpu.VMEM((SB * D,), jnp.float32),
            pltpu.VMEM((D, TAIL), jnp.float32),
            pltpu.VMEM((TAIL * D,), jnp.float32),
            pltpu.VMEM((NL * (NL + 1),), jnp.float32),
            pltpu.VMEM((NL * (NL + 1),), jnp.float32),
            pltpu.VMEM((NL * (NL + 1),), jnp.float32),
            pltpu.VMEM((NL * (NL + 1),), jnp.float32),
            pltpu.SemaphoreType.DMA,
            pltpu.SemaphoreType.DMA,
            pltpu.SemaphoreType.DMA,
            pltpu.SemaphoreType.DMA,
        ],
        compiler_params=pltpu.CompilerParams(
            use_tc_tiling_on_sc=True, needs_layout_passes=False),
    )
    flat_table = transpose_k(table.T)
    table_rm = flat_table.reshape(V, D)

    labels_flat = label_lists.reshape(-1).astype(jnp.int32)
    gather_k = pl.kernel(
        _gather_body,
        mesh=mesh,
        out_type=jax.ShapeDtypeStruct((B, D), jnp.float32),
        scratch_types=[
            pltpu.VMEM((CH * L,), jnp.int32),
            pltpu.VMEM((CH * L,), jnp.int32),
            pltpu.VMEM((CH * L, D), jnp.float32),
            pltpu.VMEM((CH * L, D), jnp.float32),
            pltpu.VMEM((CH, D), jnp.float32),
            pltpu.VMEM((CH, D), jnp.float32),
            pltpu.SemaphoreType.DMA,
            pltpu.SemaphoreType.DMA,
            pltpu.SemaphoreType.DMA,
            pltpu.SemaphoreType.DMA,
        ],
        compiler_params=pltpu.CompilerParams(use_tc_tiling_on_sc=False),
    )
    return gather_k(labels_flat, table_rm)
